# Initial kernel scaffold; baseline (speedup 1.0000x reference)
import jax
import jax.numpy as jnp
from jax.experimental import pallas as pl


def kernel(path, taxi_ids, client_ids, week, day, hour, embTaxi, embClient, embWeek, embDay, embHour, W1, b1, W2, b2):
    raise NotImplementedError("write your pallas kernel here")


# speedup vs baseline: 1.0591x; 1.0591x over previous
import functools

import jax
import jax.numpy as jnp
from jax import lax
from jax.experimental import pallas as pl
from jax.experimental.pallas import tpu as pltpu
from jax.experimental.pallas import tpu_sc as plsc

B = 16384
ED = 16
H1 = 512
NOUT = 3392

_NC = 2
_NS = 16
_NW = _NC * _NS
_BPW = B // _NW
_CHUNK = 128
_NCHUNK = _BPW // _CHUNK

_BM = 256


def _make_gather5():
    mesh = plsc.VectorSubcoreMesh(core_axis_name="c", subcore_axis_name="s")

    @functools.partial(
        pl.kernel,
        mesh=mesh,
        compiler_params=pltpu.CompilerParams(use_tc_tiling_on_sc=False),
        out_type=[jax.ShapeDtypeStruct((B, ED), jnp.float32)] * 5,
        scratch_types=[
            pltpu.VMEM((_NCHUNK, _CHUNK), jnp.int32),
            pltpu.VMEM((_BPW, ED), jnp.float32),
            pltpu.SemaphoreType.DMA,
        ],
    )
    def gather5(t0, t1, t2, t3, t4, i0, i1, i2, i3, i4,
                o0, o1, o2, o3, o4, idx_v, rows_v, sem):
        wid = lax.axis_index("s") * _NC + lax.axis_index("c")
        base = wid * _BPW
        row0 = wid * _NCHUNK
        for tab, ids, out in ((t0, i0, o0), (t1, i1, o1), (t2, i2, o2),
                              (t3, i3, o3), (t4, i4, o4)):
            pltpu.sync_copy(ids.at[pl.ds(row0, _NCHUNK)], idx_v)
            copies = [
                pltpu.async_copy(tab.at[idx_v.at[j]],
                                 rows_v.at[pl.ds(j * _CHUNK, _CHUNK)], sem)
                for j in range(_NCHUNK)
            ]
            for c in copies:
                c.wait()
            pltpu.sync_copy(rows_v, out.at[pl.ds(base, _BPW)])

    return gather5


_gather5 = _make_gather5()


def _mlp_body(path_ref, e0, e1, e2, e3, e4,
              w1p, w1t, w1c, w1w, w1d, w1h, b1, w2, b2, out_ref):
    h = jnp.dot(path_ref[...], w1p[...], preferred_element_type=jnp.float32)
    h = h + jnp.dot(e0[...], w1t[...], preferred_element_type=jnp.float32)
    h = h + jnp.dot(e1[...], w1c[...], preferred_element_type=jnp.float32)
    h = h + jnp.dot(e2[...], w1w[...], preferred_element_type=jnp.float32)
    h = h + jnp.dot(e3[...], w1d[...], preferred_element_type=jnp.float32)
    h = h + jnp.dot(e4[...], w1h[...], preferred_element_type=jnp.float32)
    h = jnp.maximum(h + b1[...], 0.0)
    logits = jnp.dot(h, w2[...], preferred_element_type=jnp.float32) + b2[...]
    m = jnp.max(logits, axis=1, keepdims=True)
    e = jnp.exp(logits - m)
    out_ref[...] = e / jnp.sum(e, axis=1, keepdims=True)


def _mlp(path, embs, w1p, w1ts, b1, w2, b2):
    grid = (B // _BM,)
    const = lambda shape: pl.BlockSpec(shape, lambda i: (0, 0))
    in_specs = [pl.BlockSpec((_BM, path.shape[1]), lambda i: (i, 0))]
    in_specs += [pl.BlockSpec((_BM, ED), lambda i: (i, 0)) for _ in range(5)]
    in_specs += [const((w1p.shape[0], H1))]
    in_specs += [const((ED, H1)) for _ in range(5)]
    in_specs += [const((1, H1)), const((H1, NOUT)), const((1, NOUT))]
    return pl.pallas_call(
        _mlp_body,
        grid=grid,
        in_specs=in_specs,
        out_specs=pl.BlockSpec((_BM, NOUT), lambda i: (i, 0)),
        out_shape=jax.ShapeDtypeStruct((B, NOUT), jnp.float32),
    )(path, *embs, w1p, *w1ts, b1, w2, b2)


def kernel(path, taxi_ids, client_ids, week, day, hour,
           embTaxi, embClient, embWeek, embDay, embHour, W1, b1, W2, b2):
    pad_tab = lambda t: jnp.pad(t, ((0, 0), (0, ED - t.shape[1])))
    tables = [pad_tab(t) for t in (embTaxi, embClient, embWeek, embDay, embHour)]
    ids = [i.astype(jnp.int32).reshape(B // _CHUNK, _CHUNK)
           for i in (taxi_ids, client_ids, week, day, hour)]
    embs = _gather5(*tables, *ids)

    npath = path.shape[1]
    w1p = jnp.pad(W1[:npath], ((0, 0), (0, H1 - W1.shape[1])))
    w1ts = [jnp.pad(W1[npath + 10 * t: npath + 10 * (t + 1)],
                    ((0, ED - 10), (0, H1 - W1.shape[1]))) for t in range(5)]
    b1p = jnp.pad(b1, (0, H1 - b1.shape[0])).reshape(1, H1)
    w2p = jnp.pad(W2, ((0, H1 - W2.shape[0]), (0, 0)))
    b2p = b2.reshape(1, NOUT)
    return _mlp(path, embs, w1p, w1ts, b1p, w2p, b2p)


# speedup vs baseline: 1.0714x; 1.0116x over previous
import functools

import jax
import jax.numpy as jnp
from jax import lax
from jax.experimental import pallas as pl
from jax.experimental.pallas import tpu as pltpu
from jax.experimental.pallas import tpu_sc as plsc

B = 16384
ED = 16
H1 = 512
NOUT = 3392

_NC = 2
_NS = 16
_NW = _NC * _NS
_BPW = B // _NW
_CHUNK = 128
_NCHUNK = _BPW // _CHUNK

_BM = 256


def _make_gather5():
    mesh = plsc.VectorSubcoreMesh(core_axis_name="c", subcore_axis_name="s")

    @functools.partial(
        pl.kernel,
        mesh=mesh,
        compiler_params=pltpu.CompilerParams(use_tc_tiling_on_sc=False),
        out_type=[jax.ShapeDtypeStruct((B, ED), jnp.float32)] * 5,
        scratch_types=[
            pltpu.VMEM((5, _BPW), jnp.int32),
            pltpu.VMEM((5, _BPW, ED), jnp.float32),
            pltpu.SemaphoreType.DMA,
        ],
    )
    def gather5(t0, t1, t2, t3, t4, i0, i1, i2, i3, i4,
                o0, o1, o2, o3, o4, idx_v, rows_v, sem):
        wid = lax.axis_index("s") * _NC + lax.axis_index("c")
        base = wid * _BPW
        tabs = (t0, t1, t2, t3, t4)
        idss = (i0, i1, i2, i3, i4)
        outs = (o0, o1, o2, o3, o4)
        idx_copies = [
            pltpu.async_copy(idss[t].at[pl.ds(base, _BPW)], idx_v.at[t], sem)
            for t in range(5)
        ]
        for c in idx_copies:
            c.wait()
        gathers = [
            pltpu.async_copy(tabs[t].at[idx_v.at[t, pl.ds(j * _CHUNK, _CHUNK)]],
                             rows_v.at[t, pl.ds(j * _CHUNK, _CHUNK)], sem)
            for t in range(5) for j in range(_NCHUNK)
        ]
        for c in gathers:
            c.wait()
        wb = [
            pltpu.async_copy(rows_v.at[t], outs[t].at[pl.ds(base, _BPW)], sem)
            for t in range(5)
        ]
        for c in wb:
            c.wait()

    return gather5


_gather5 = _make_gather5()


def _mlp_body(path_ref, e0, e1, e2, e3, e4,
              w1p, w1t, w1c, w1w, w1d, w1h, b1, w2, b2, out_ref):
    h = jnp.dot(path_ref[...], w1p[...], preferred_element_type=jnp.float32)
    h = h + jnp.dot(e0[...], w1t[...], preferred_element_type=jnp.float32)
    h = h + jnp.dot(e1[...], w1c[...], preferred_element_type=jnp.float32)
    h = h + jnp.dot(e2[...], w1w[...], preferred_element_type=jnp.float32)
    h = h + jnp.dot(e3[...], w1d[...], preferred_element_type=jnp.float32)
    h = h + jnp.dot(e4[...], w1h[...], preferred_element_type=jnp.float32)
    h = jnp.maximum(h + b1[...], 0.0)
    logits = jnp.dot(h.astype(jnp.bfloat16), w2[...],
                     preferred_element_type=jnp.float32) + b2[...]
    m = jnp.max(logits, axis=1, keepdims=True)
    e = jnp.exp(logits - m)
    out_ref[...] = e / jnp.sum(e, axis=1, keepdims=True)


def _mlp(path, embs, w1p, w1ts, b1, w2, b2):
    grid = (B // _BM,)
    const = lambda shape: pl.BlockSpec(shape, lambda i: (0, 0))
    in_specs = [pl.BlockSpec((_BM, path.shape[1]), lambda i: (i, 0))]
    in_specs += [pl.BlockSpec((_BM, ED), lambda i: (i, 0)) for _ in range(5)]
    in_specs += [const((w1p.shape[0], H1))]
    in_specs += [const((ED, H1)) for _ in range(5)]
    in_specs += [const((1, H1)), const((H1, NOUT)), const((1, NOUT))]
    return pl.pallas_call(
        _mlp_body,
        grid=grid,
        in_specs=in_specs,
        out_specs=pl.BlockSpec((_BM, NOUT), lambda i: (i, 0)),
        out_shape=jax.ShapeDtypeStruct((B, NOUT), jnp.float32),
    )(path, *embs, w1p, *w1ts, b1, w2, b2)


def kernel(path, taxi_ids, client_ids, week, day, hour,
           embTaxi, embClient, embWeek, embDay, embHour, W1, b1, W2, b2):
    pad_tab = lambda t: jnp.pad(t, ((0, 0), (0, ED - t.shape[1])))
    tables = [pad_tab(t) for t in (embTaxi, embClient, embWeek, embDay, embHour)]
    ids = [i.astype(jnp.int32)
           for i in (taxi_ids, client_ids, week, day, hour)]
    embs = _gather5(*tables, *ids)

    npath = path.shape[1]
    w1p = jnp.pad(W1[:npath], ((0, 0), (0, H1 - W1.shape[1])))
    w1ts = [jnp.pad(W1[npath + 10 * t: npath + 10 * (t + 1)],
                    ((0, ED - 10), (0, H1 - W1.shape[1]))) for t in range(5)]
    b1p = jnp.pad(b1, (0, H1 - b1.shape[0])).reshape(1, H1)
    w2p = jnp.pad(W2, ((0, H1 - W2.shape[0]), (0, 0))).astype(jnp.bfloat16)
    b2p = b2.reshape(1, NOUT)
    return _mlp(path, embs, w1p, w1ts, b1p, w2p, b2p)
